# Initial kernel scaffold; baseline (speedup 1.0000x reference)
import functools

import jax
import jax.numpy as jnp
from jax import lax
from jax.experimental import pallas as pl
from jax.experimental.pallas import tpu as pltpu
from jax.experimental.pallas import tpu_sc as plsc

B, L, F = 16384, 50, 768
EMBED_DIM, H1, H2 = 128, 256, 128
LPAD = 64
NC, NS = 2, 16
NW = NC * NS
BAGS_PER_W = B // NW
NB = 64
NCHUNK = BAGS_PER_W // NB

_mesh = plsc.VectorSubcoreMesh(core_axis_name="c", subcore_axis_name="s")


@functools.partial(
    pl.kernel,
    out_type=jax.ShapeDtypeStruct((B * F,), jnp.float32),
    mesh=_mesh,
    scratch_types=[
        pltpu.VMEM((NB * LPAD,), jnp.int32),
        pltpu.VMEM((NB * F,), jnp.float32),
    ],
)
def _counts_kernel(idx_hbm, counts_hbm, idx_v, counts_v):
    wid = lax.axis_index("s") * NC + lax.axis_index("c")
    base_bag = wid * BAGS_PER_W

    ones = jnp.ones((16,), jnp.float32)
    zeros16 = jnp.zeros((16,), jnp.float32)
    lane = lax.iota(jnp.int32, 16)
    tail_mask = lane < (L - 48)

    def _z(i, c):
        counts_v[pl.ds(i * 16, 16)] = zeros16
        return c
    lax.fori_loop(0, NB * F // 16, _z, 0)

    def _chunk(chunk, carry):
        row0 = base_bag + chunk * NB
        pltpu.sync_copy(idx_hbm.at[pl.ds(row0 * LPAD, NB * LPAD)], idx_v)

        def _bag(b, c):
            off = b * LPAD
            cbase = b * F
            for j in range(4):
                iv = idx_v[pl.ds(off + j * 16, 16)] + cbase
                if j < 3:
                    plsc.addupdate_scatter(counts_v, [iv], ones)
                else:
                    plsc.addupdate_scatter(counts_v, [iv], ones, mask=tail_mask)
            return c
        lax.fori_loop(0, NB, _bag, 0)

        pltpu.sync_copy(counts_v, counts_hbm.at[pl.ds(row0 * F, NB * F)])

        def _unbag(b, c):
            off = b * LPAD
            cbase = b * F
            for j in range(4):
                iv = idx_v[pl.ds(off + j * 16, 16)] + cbase
                if j < 3:
                    plsc.store_scatter(counts_v, [iv], zeros16)
                else:
                    plsc.store_scatter(counts_v, [iv], zeros16, mask=tail_mask)
            return c
        lax.fori_loop(0, NB, _unbag, 0)
        return carry

    lax.fori_loop(0, NCHUNK, _chunk, 0)


def _m1_body(emb_ref, w1t_ref, m1_ref):
    m1_ref[...] = jnp.dot(emb_ref[...], w1t_ref[...],
                          preferred_element_type=jnp.float32)


def _mlp_body(counts_ref, m1_ref, b1_ref, w2t_ref, b2_ref, w3t_ref, b3_ref,
              out_ref):
    h1 = jnp.dot(counts_ref[...], m1_ref[...],
                 preferred_element_type=jnp.float32) + b1_ref[...]
    h1 = jnp.maximum(h1, 0.0)
    h2 = jnp.dot(h1, w2t_ref[...], preferred_element_type=jnp.float32) \
        + b2_ref[...]
    h2 = jnp.maximum(h2, 0.0)
    out_ref[...] = jnp.dot(h2, w3t_ref[...],
                           preferred_element_type=jnp.float32) + b3_ref[...]


_MLP_BLK = 2048


def kernel(features_indices, emb, W1, b1, W2, b2, W3, b3):
    idx = features_indices.astype(jnp.int32)
    idx_pad = jnp.pad(idx, ((0, 0), (0, LPAD - L))).reshape(-1)

    counts = _counts_kernel(idx_pad).reshape(B, F)

    m1 = pl.pallas_call(
        _m1_body,
        out_shape=jax.ShapeDtypeStruct((F, H1), jnp.float32),
    )(emb, W1.T)

    out = pl.pallas_call(
        _mlp_body,
        grid=(B // _MLP_BLK,),
        in_specs=[
            pl.BlockSpec((_MLP_BLK, F), lambda i: (i, 0)),
            pl.BlockSpec((F, H1), lambda i: (0, 0)),
            pl.BlockSpec((1, H1), lambda i: (0, 0)),
            pl.BlockSpec((H1, H2), lambda i: (0, 0)),
            pl.BlockSpec((1, H2), lambda i: (0, 0)),
            pl.BlockSpec((H2, 1), lambda i: (0, 0)),
            pl.BlockSpec((1, 1), lambda i: (0, 0)),
        ],
        out_specs=pl.BlockSpec((_MLP_BLK, 1), lambda i: (i, 0)),
        out_shape=jax.ShapeDtypeStruct((B, 1), jnp.float32),
    )(counts, m1, b1.reshape(1, H1), W2.T, b2.reshape(1, H2), W3.T,
      b3.reshape(1, 1))
    return out


# speedup vs baseline: 20.0056x; 20.0056x over previous
import functools

import jax
import jax.numpy as jnp
from jax import lax
from jax.experimental import pallas as pl
from jax.experimental.pallas import tpu as pltpu
from jax.experimental.pallas import tpu_sc as plsc

B, L, F = 16384, 50, 768
EMBED_DIM, H1, H2 = 128, 256, 128
LPAD = 64
NC, NS = 2, 16
NW = NC * NS
BAGS_PER_W = B // NW
NB = 64
NCHUNK = BAGS_PER_W // NB

_mesh = plsc.VectorSubcoreMesh(core_axis_name="c", subcore_axis_name="s")


@functools.partial(
    pl.kernel,
    out_type=jax.ShapeDtypeStruct((B * F,), jnp.float32),
    mesh=_mesh,
    scratch_types=[
        pltpu.VMEM((NB * LPAD,), jnp.int32),
        pltpu.VMEM((NB * F,), jnp.float32),
    ],
    compiler_params=pltpu.CompilerParams(needs_layout_passes=False),
)
def _counts_kernel(idx_hbm, counts_hbm, idx_v, counts_v):
    wid = lax.axis_index("s") * NC + lax.axis_index("c")
    base_bag = wid * BAGS_PER_W

    ones = jnp.ones((16,), jnp.float32)
    zeros16 = jnp.zeros((16,), jnp.float32)
    lane = lax.iota(jnp.int32, 16)
    tail_mask = lane < (L - 48)

    def _z(i, c):
        counts_v[pl.ds(i * 16, 16)] = zeros16
        return c
    lax.fori_loop(0, NB * F // 16, _z, 0)

    def _chunk(chunk, carry):
        row0 = base_bag + chunk * NB
        pltpu.sync_copy(idx_hbm.at[pl.ds(row0 * LPAD, NB * LPAD)], idx_v)

        def _bag(b, c):
            off = b * LPAD
            cbase = b * F
            for j in range(4):
                iv = idx_v[pl.ds(off + j * 16, 16)] + cbase
                if j < 3:
                    plsc.addupdate_scatter(counts_v, [iv], ones)
                else:
                    plsc.addupdate_scatter(counts_v, [iv], ones, mask=tail_mask)
            return c
        lax.fori_loop(0, NB, _bag, 0)

        pltpu.sync_copy(counts_v, counts_hbm.at[pl.ds(row0 * F, NB * F)])

        def _unbag(b, c):
            off = b * LPAD
            cbase = b * F
            for j in range(4):
                iv = idx_v[pl.ds(off + j * 16, 16)] + cbase
                if j < 3:
                    plsc.store_scatter(counts_v, [iv], zeros16)
                else:
                    plsc.store_scatter(counts_v, [iv], zeros16, mask=tail_mask)
            return c
        lax.fori_loop(0, NB, _unbag, 0)
        return carry

    lax.fori_loop(0, NCHUNK, _chunk, 0)


def _m1_body(emb_ref, w1t_ref, m1_ref):
    m1_ref[...] = jnp.dot(emb_ref[...], w1t_ref[...],
                          preferred_element_type=jnp.float32)


def _mlp_body(counts_ref, m1_ref, b1_ref, w2t_ref, b2_ref, w3t_ref, b3_ref,
              out_ref):
    h1 = jnp.dot(counts_ref[...], m1_ref[...],
                 preferred_element_type=jnp.float32) + b1_ref[...]
    h1 = jnp.maximum(h1, 0.0)
    h2 = jnp.dot(h1, w2t_ref[...], preferred_element_type=jnp.float32) \
        + b2_ref[...]
    h2 = jnp.maximum(h2, 0.0)
    out_ref[...] = jnp.dot(h2, w3t_ref[...],
                           preferred_element_type=jnp.float32) + b3_ref[...]


_MLP_BLK = 2048


def kernel(features_indices, emb, W1, b1, W2, b2, W3, b3):
    idx = features_indices.astype(jnp.int32)
    idx_pad = jnp.pad(idx, ((0, 0), (0, LPAD - L))).reshape(-1)

    counts = _counts_kernel(idx_pad).reshape(B, F)

    m1 = pl.pallas_call(
        _m1_body,
        out_shape=jax.ShapeDtypeStruct((F, H1), jnp.float32),
    )(emb, W1.T)

    out = pl.pallas_call(
        _mlp_body,
        grid=(B // _MLP_BLK,),
        in_specs=[
            pl.BlockSpec((_MLP_BLK, F), lambda i: (i, 0)),
            pl.BlockSpec((F, H1), lambda i: (0, 0)),
            pl.BlockSpec((1, H1), lambda i: (0, 0)),
            pl.BlockSpec((H1, H2), lambda i: (0, 0)),
            pl.BlockSpec((1, H2), lambda i: (0, 0)),
            pl.BlockSpec((H2, 1), lambda i: (0, 0)),
            pl.BlockSpec((1, 1), lambda i: (0, 0)),
        ],
        out_specs=pl.BlockSpec((_MLP_BLK, 1), lambda i: (i, 0)),
        out_shape=jax.ShapeDtypeStruct((B, 1), jnp.float32),
    )(counts, m1, b1.reshape(1, H1), W2.T, b2.reshape(1, H2), W3.T,
      b3.reshape(1, 1))
    return out


# speedup vs baseline: 20.4127x; 1.0203x over previous
import functools

import jax
import jax.numpy as jnp
from jax import lax
from jax.experimental import pallas as pl
from jax.experimental.pallas import tpu as pltpu
from jax.experimental.pallas import tpu_sc as plsc

B, L, F = 16384, 50, 768
EMBED_DIM, H1, H2 = 128, 256, 128
NC, NS = 2, 16
NW = NC * NS
BAGS_PER_W = B // NW
NB = 64
NCHUNK = BAGS_PER_W // NB
IDXW = NB * L + 16

_mesh = plsc.VectorSubcoreMesh(core_axis_name="c", subcore_axis_name="s")


@functools.partial(
    pl.kernel,
    out_type=jax.ShapeDtypeStruct((B * F,), jnp.float32),
    mesh=_mesh,
    scratch_types=[
        pltpu.VMEM((IDXW,), jnp.int32),
        pltpu.VMEM((IDXW,), jnp.int32),
        pltpu.VMEM((NB * F,), jnp.float32),
        pltpu.VMEM((NB * F,), jnp.float32),
        pltpu.SemaphoreType.DMA,
        pltpu.SemaphoreType.DMA,
    ],
    compiler_params=pltpu.CompilerParams(needs_layout_passes=False),
)
def _counts_kernel(idx_hbm, counts_hbm, idx_v0, idx_v1, counts_v0, counts_v1,
                   sem0, sem1):
    wid = lax.axis_index("s") * NC + lax.axis_index("c")
    base_bag = wid * BAGS_PER_W
    idx_bufs = (idx_v0, idx_v1)
    cnt_bufs = (counts_v0, counts_v1)
    sems = (sem0, sem1)

    ones = jnp.ones((16,), jnp.float32)
    zeros16 = jnp.zeros((16,), jnp.float32)
    lane = lax.iota(jnp.int32, 16)
    tail_mask = lane < (L - 48)

    for buf in range(2):
        def _z(i, c, buf=buf):
            cnt_bufs[buf][pl.ds(i * 16, 16)] = zeros16
            return c
        lax.fori_loop(0, NB * F // 16, _z, 0)
        idx_bufs[buf][pl.ds(NB * L, 16)] = lane * 0

    def _scatter(buf, add):
        val = ones if add else zeros16
        op = plsc.addupdate_scatter if add else plsc.store_scatter
        idx_v, counts_v = idx_bufs[buf], cnt_bufs[buf]

        def _bag(b, c):
            off = b * L
            cbase = b * F
            for j in range(4):
                iv = idx_v[pl.ds(off + j * 16, 16)] + cbase
                if j < 3:
                    op(counts_v, [iv], val)
                else:
                    op(counts_v, [iv], val, mask=tail_mask)
            return c
        lax.fori_loop(0, NB, _bag, 0)

    out_dma = [None, None]
    for chunk in range(NCHUNK):
        buf = chunk % 2
        row0 = base_bag + chunk * NB
        if out_dma[buf] is not None:
            out_dma[buf].wait()
            _scatter(buf, add=False)
        pltpu.sync_copy(idx_hbm.at[pl.ds(row0 * L, NB * L)],
                        idx_bufs[buf].at[pl.ds(0, NB * L)])
        _scatter(buf, add=True)
        out_dma[buf] = pltpu.async_copy(
            cnt_bufs[buf], counts_hbm.at[pl.ds(row0 * F, NB * F)],
            sems[buf])
    out_dma[0].wait()
    out_dma[1].wait()


def _m1_body(emb_ref, w1t_ref, m1_ref):
    m1_ref[...] = jnp.dot(emb_ref[...], w1t_ref[...],
                          preferred_element_type=jnp.float32)


def _mlp_body(counts_ref, m1_ref, b1_ref, w2t_ref, b2_ref, w3t_ref, b3_ref,
              out_ref):
    h1 = jnp.dot(counts_ref[...], m1_ref[...],
                 preferred_element_type=jnp.float32) + b1_ref[...]
    h1 = jnp.maximum(h1, 0.0)
    h2 = jnp.dot(h1, w2t_ref[...], preferred_element_type=jnp.float32) \
        + b2_ref[...]
    h2 = jnp.maximum(h2, 0.0)
    out_ref[...] = jnp.dot(h2, w3t_ref[...],
                           preferred_element_type=jnp.float32) + b3_ref[...]


_MLP_BLK = 2048


def kernel(features_indices, emb, W1, b1, W2, b2, W3, b3):
    idx = features_indices.astype(jnp.int32).reshape(-1)

    counts = _counts_kernel(idx).reshape(B, F)

    m1 = pl.pallas_call(
        _m1_body,
        out_shape=jax.ShapeDtypeStruct((F, H1), jnp.float32),
    )(emb, W1.T)

    out = pl.pallas_call(
        _mlp_body,
        grid=(B // _MLP_BLK,),
        in_specs=[
            pl.BlockSpec((_MLP_BLK, F), lambda i: (i, 0)),
            pl.BlockSpec((F, H1), lambda i: (0, 0)),
            pl.BlockSpec((1, H1), lambda i: (0, 0)),
            pl.BlockSpec((H1, H2), lambda i: (0, 0)),
            pl.BlockSpec((1, H2), lambda i: (0, 0)),
            pl.BlockSpec((H2, 1), lambda i: (0, 0)),
            pl.BlockSpec((1, 1), lambda i: (0, 0)),
        ],
        out_specs=pl.BlockSpec((_MLP_BLK, 1), lambda i: (i, 0)),
        out_shape=jax.ShapeDtypeStruct((B, 1), jnp.float32),
    )(counts, m1, b1.reshape(1, H1), W2.T, b2.reshape(1, H2), W3.T,
      b3.reshape(1, 1))
    return out


# speedup vs baseline: 28.0763x; 1.3754x over previous
import functools

import jax
import jax.numpy as jnp
from jax import lax
from jax.experimental import pallas as pl
from jax.experimental.pallas import tpu as pltpu
from jax.experimental.pallas import tpu_sc as plsc

B, L, F = 16384, 50, 768
EMBED_DIM, H1, H2 = 128, 256, 128
NPART = F // 128
NC, NS = 2, 16
NW = NC * NS
BAGS_PER_W = B // NW
NB = 64
NCHUNK = BAGS_PER_W // NB
NIB = 4
IDXW = NB * L + 16
PSTRIDE = NB * 128

_mesh = plsc.VectorSubcoreMesh(core_axis_name="c", subcore_axis_name="s")


@functools.partial(
    pl.kernel,
    out_type=jax.ShapeDtypeStruct((NPART * B * 128,), jnp.float32),
    mesh=_mesh,
    scratch_types=[
        [pltpu.VMEM((IDXW,), jnp.int32) for _ in range(NIB)],
        [pltpu.VMEM((NB * F,), jnp.float32) for _ in range(2)],
        [pltpu.SemaphoreType.DMA for _ in range(NIB + 2)],
    ],
    compiler_params=pltpu.CompilerParams(needs_layout_passes=False),
)
def _counts_kernel(idx_hbm, counts_hbm, idx_bufs, cnt_bufs, sems):
    wid = lax.axis_index("s") * NC + lax.axis_index("c")
    base_bag = wid * BAGS_PER_W

    ones = jnp.ones((16,), jnp.float32)
    zeros16 = jnp.zeros((16,), jnp.float32)
    lane = lax.iota(jnp.int32, 16)
    tail_mask = lane < (L - 48)

    for buf in range(2):
        def _z(i, c, buf=buf):
            cnt_bufs[buf][pl.ds(i * 16, 16)] = zeros16
            return c
        lax.fori_loop(0, NB * F // 16, _z, 0)
    for ib in range(NIB):
        idx_bufs[ib][pl.ds(NB * L, 16)] = lane * 0

    def _idx_dma(chunk):
        row0 = base_bag + chunk * NB
        return pltpu.async_copy(
            idx_hbm.at[pl.ds(row0 * L, NB * L)],
            idx_bufs[chunk % NIB].at[pl.ds(0, NB * L)],
            sems[chunk % NIB])

    def _scatter(chunk, add):
        val = ones if add else zeros16
        op = plsc.addupdate_scatter if add else plsc.store_scatter
        idx_v, counts_v = idx_bufs[chunk % NIB], cnt_bufs[chunk % 2]

        def _bag(b, c):
            off = b * L
            bbase = b * 128
            for j in range(4):
                iv = idx_v[pl.ds(off + j * 16, 16)]
                dst = ((iv >> 7) << 13) + ((iv & 127) + bbase)
                if j < 3:
                    op(counts_v, [dst], val)
                else:
                    op(counts_v, [dst], val, mask=tail_mask)
            return c
        lax.fori_loop(0, NB, _bag, 0)

    def _out_dma(chunk):
        row0 = base_bag + chunk * NB
        buf = chunk % 2
        return [pltpu.async_copy(
            cnt_bufs[buf].at[pl.ds(p * PSTRIDE, PSTRIDE)],
            counts_hbm.at[pl.ds(p * B * 128 + row0 * 128, PSTRIDE)],
            sems[NIB + buf]) for p in range(NPART)]

    out_dma = [None, None]
    idx_dma = _idx_dma(0)
    for chunk in range(NCHUNK):
        buf = chunk % 2
        if chunk + 1 < NCHUNK:
            next_idx_dma = _idx_dma(chunk + 1)
        if out_dma[buf] is not None:
            for cp in out_dma[buf]:
                cp.wait()
            _scatter(chunk - 2, add=False)
        idx_dma.wait()
        _scatter(chunk, add=True)
        out_dma[buf] = _out_dma(chunk)
        if chunk + 1 < NCHUNK:
            idx_dma = next_idx_dma
    for buf in range(2):
        for cp in out_dma[buf]:
            cp.wait()


def _m1_body(emb_ref, w1t_ref, m1_ref):
    m1_ref[...] = jnp.dot(emb_ref[...], w1t_ref[...],
                          preferred_element_type=jnp.float32)


def _mlp_body(c0, c1, c2, c3, c4, c5, m10, m11, m12, m13, m14, m15,
              b1_ref, w2t_ref, b2_ref, w3t_ref, b3_ref, out_ref):
    cs = (c0, c1, c2, c3, c4, c5)
    ms = (m10, m11, m12, m13, m14, m15)
    h1 = b1_ref[...]
    for p in range(NPART):
        h1 = h1 + jnp.dot(cs[p][...], ms[p][...],
                          preferred_element_type=jnp.float32)
    h1 = jnp.maximum(h1, 0.0)
    h2 = jnp.dot(h1, w2t_ref[...], preferred_element_type=jnp.float32) \
        + b2_ref[...]
    h2 = jnp.maximum(h2, 0.0)
    out_ref[...] = jnp.dot(h2, w3t_ref[...],
                           preferred_element_type=jnp.float32) + b3_ref[...]


_MLP_BLK = 2048


def kernel(features_indices, emb, W1, b1, W2, b2, W3, b3):
    idx = features_indices.astype(jnp.int32).reshape(-1)

    counts_pm = _counts_kernel(idx).reshape(NPART * B, 128)

    m1 = pl.pallas_call(
        _m1_body,
        out_shape=jax.ShapeDtypeStruct((F, H1), jnp.float32),
    )(emb, W1.T)

    nblk = B // _MLP_BLK
    cnt_specs = [
        pl.BlockSpec((_MLP_BLK, 128), lambda i, p=p: (p * nblk + i, 0))
        for p in range(NPART)
    ]
    m1_specs = [
        pl.BlockSpec((128, H1), lambda i, p=p: (p, 0)) for p in range(NPART)
    ]
    out = pl.pallas_call(
        _mlp_body,
        grid=(nblk,),
        in_specs=cnt_specs + m1_specs + [
            pl.BlockSpec((1, H1), lambda i: (0, 0)),
            pl.BlockSpec((H1, H2), lambda i: (0, 0)),
            pl.BlockSpec((1, H2), lambda i: (0, 0)),
            pl.BlockSpec((H2, 1), lambda i: (0, 0)),
            pl.BlockSpec((1, 1), lambda i: (0, 0)),
        ],
        out_specs=pl.BlockSpec((_MLP_BLK, 1), lambda i: (i, 0)),
        out_shape=jax.ShapeDtypeStruct((B, 1), jnp.float32),
    )(*([counts_pm] * NPART), *([m1] * NPART),
      b1.reshape(1, H1), W2.T, b2.reshape(1, H2), W3.T, b3.reshape(1, 1))
    return out
